# baseline (speedup 1.0000x reference)
import functools

import jax
import jax.numpy as jnp
from jax import lax
from jax.experimental import pallas as pl
from jax.experimental.pallas import tpu as pltpu
from jax.experimental.pallas import tpu_sc as plsc

N = 10000
E = 320000
DH = 128
DO = 32
NC = 2
NS = 16
L = 16
NPAD = 10240
RPT = NPAD // NS
W = 125
EPT = E // NS
NWIN = EPT // W
G = 4
BW = 40
WCH = 80
RCH = RPT // WCH

_MESH = plsc.VectorSubcoreMesh(core_axis_name="c", subcore_axis_name="s")


def _splat(ref, i):
    return plsc.load_gather(ref, [jnp.zeros((L,), jnp.int32) + i])


def _zero_rows(ref, nrow, ncol):
    def body(i, _):
        for j in range(ncol // L):
            ref[i, pl.ds(j * L, L)] = jnp.zeros((L,), jnp.float32)
        return 0
    lax.fori_loop(0, nrow, body, 0)


_DEG_WPT = E // NC // NS // W


def _deg_body(coli, ew, deg_out, coli_v, ew_v, zd_v, deg_sh, sem):
    c = lax.axis_index("c")
    s = lax.axis_index("s")
    def zb(i, _):
        zd_v[pl.ds(i * L, L)] = jnp.zeros((L,), jnp.float32)
        return 0
    lax.fori_loop(0, RPT // L, zb, 0)
    pltpu.sync_copy(zd_v, deg_sh.at[pl.ds(s * RPT, RPT)])
    wbase = (c * NS + s) * _DEG_WPT
    pltpu.sync_copy(coli.at[pl.ds(wbase, _DEG_WPT)], coli_v)
    pltpu.sync_copy(ew.at[pl.ds(wbase, _DEG_WPT)], ew_v)
    plsc.subcore_barrier()
    def grp(g, _):
        ds_ = [pltpu.async_copy(ew_v.at[g * 8 + k],
                                deg_sh.at[coli_v.at[g * 8 + k]], sem,
                                add=True)
               for k in range(8)]
        for d in ds_:
            d.wait()
        return 0
    lax.fori_loop(0, _DEG_WPT // 8, grp, 0)
    plsc.subcore_barrier()
    pltpu.sync_copy(deg_sh.at[pl.ds(s * RPT, RPT)],
                    deg_out.at[c, pl.ds(s * RPT, RPT)])


def _sc_deg(coli, ew):
    f = pl.kernel(
        _deg_body,
        out_type=jax.ShapeDtypeStruct((NC, NPAD), jnp.float32),
        mesh=_MESH,
        compiler_params=pltpu.CompilerParams(needs_layout_passes=False, use_tc_tiling_on_sc=False),
        scratch_types=[
            pltpu.VMEM((_DEG_WPT, W), jnp.int32),
            pltpu.VMEM((_DEG_WPT, W), jnp.float32),
            pltpu.VMEM((RPT,), jnp.float32),
            pltpu.VMEM_SHARED((NPAD,), jnp.float32),
            pltpu.SemaphoreType.DMA,
        ],
    )
    return f(coli, ew)


def _conv_body(d2, do_relu, gdep, stage, table, rowi, coli, ew, b, dis_in, out,
               *refs):
    c = lax.axis_index("c")
    s = lax.axis_index("s")
    if stage:
        (rowi_v, coli_v, ew_v, *rbufs, b_v, disc_v, t_sh, acc_sh,
         gsem, ssem) = refs
    else:
        (rowi_v, coli_v, ew_v, *rbufs, b_v, disc_v, acc_sh,
         gsem, ssem) = refs
        t_sh = None
    rb0, rb1 = rbufs[0], rbufs[1]

    _zero_rows(rb0, WCH, d2)
    for k in range(RCH):
        pltpu.sync_copy(rb0.at[pl.ds(0, WCH)], acc_sh.at[pl.ds(s * RPT + k * WCH, WCH)])
    pltpu.sync_copy(b.at[pl.ds(c * d2, d2)], b_v)
    if stage:
        @pl.when(s == 0)
        def _():
            pltpu.sync_copy(table.at[c], t_sh)
    plsc.subcore_barrier()

    def block(blk, _):
        wbase = s * NWIN + blk * BW
        pltpu.sync_copy(rowi.at[pl.ds(wbase, BW)], rowi_v)
        pltpu.sync_copy(coli.at[pl.ds(wbase, BW)], coli_v)
        pltpu.sync_copy(ew.at[pl.ds(wbase, BW)], ew_v)
        def group(g, _):
            if stage:
                gd = [pltpu.async_copy(t_sh.at[rowi_v.at[g * gdep + k]],
                                       rbufs[k], gsem)
                      for k in range(gdep)]
            else:
                gd = [pltpu.async_copy(table.at[c].at[rowi_v.at[g * gdep + k]],
                                       rbufs[k], gsem)
                      for k in range(gdep)]
            sd = []
            for k in range(gdep):
                w = g * gdep + k
                gd[k].wait()
                def scale(i, _):
                    for u in range(25):
                        e = i * 25 + u
                        sp = plsc.load_gather(
                            ew_v, [jnp.zeros((L,), jnp.int32) + w,
                                   jnp.zeros((L,), jnp.int32) + e])
                        for j in range(d2 // L):
                            rbufs[k][e, pl.ds(j * L, L)] = (
                                rbufs[k][e, pl.ds(j * L, L)] * sp)
                    return 0
                lax.fori_loop(0, W // 25, scale, 0)
                sd.append(pltpu.async_copy(rbufs[k], acc_sh.at[coli_v.at[w]],
                                           ssem, add=True))
            for d in sd:
                d.wait()
            return 0
        lax.fori_loop(0, BW // gdep, group, 0)
        return 0
    lax.fori_loop(0, NWIN // BW, block, 0)
    plsc.subcore_barrier()

    def epi(k, _):
        start = s * RPT + k * WCH
        @pl.when(start < N)
        def _():
            pltpu.sync_copy(acc_sh.at[pl.ds(start, WCH)], rb0.at[pl.ds(0, WCH)])
            if stage:
                pltpu.sync_copy(t_sh.at[pl.ds(start, WCH)], rb1.at[pl.ds(0, WCH)])
            else:
                pltpu.sync_copy(table.at[c, pl.ds(start, WCH)], rb1.at[pl.ds(0, WCH)])
            pltpu.sync_copy(dis_in.at[pl.ds(start, WCH)], disc_v)
            def erow(r, _):
                dsp = _splat(disc_v, r)
                for j in range(d2 // L):
                    a = rb0[r, pl.ds(j * L, L)]
                    t = rb1[r, pl.ds(j * L, L)]
                    bb = b_v[pl.ds(j * L, L)]
                    v = dsp * (a + t) + bb
                    if do_relu:
                        v = jnp.maximum(v, 0.0)
                    rb0[r, pl.ds(j * L, L)] = v
                return 0
            lax.fori_loop(0, WCH, erow, 0)
            pltpu.sync_copy(rb0.at[pl.ds(0, WCH)], out.at[c, pl.ds(start, WCH)])
        return 0
    lax.fori_loop(0, RCH, epi, 0)


def _sc_conv(table, rowi, coli, ew, b, dis, d2, do_relu, gdep, stage):
    scratch = [
        pltpu.VMEM((BW, W), jnp.int32),
        pltpu.VMEM((BW, W), jnp.int32),
        pltpu.VMEM((BW, W), jnp.float32),
    ]
    scratch += [pltpu.VMEM((W, d2), jnp.float32) for _ in range(gdep)]
    scratch += [
        pltpu.VMEM((d2,), jnp.float32),
        pltpu.VMEM((WCH,), jnp.float32),
    ]
    if stage:
        scratch.append(pltpu.VMEM_SHARED((N, d2), jnp.float32))
    scratch += [
        pltpu.VMEM_SHARED((NPAD, d2), jnp.float32),
        pltpu.SemaphoreType.DMA,
        pltpu.SemaphoreType.DMA,
    ]
    f = pl.kernel(
        functools.partial(_conv_body, d2, do_relu, gdep, stage),
        out_type=jax.ShapeDtypeStruct((NC, N, d2), jnp.float32),
        mesh=_MESH,
        compiler_params=pltpu.CompilerParams(needs_layout_passes=False, use_tc_tiling_on_sc=False),
        scratch_types=scratch,
    )
    return f(table, rowi, coli, ew, b, dis)


_RB = 1000


def _mm1_k(x_ref, w_ref, deg_ref, o_ref, dis_ref):
    dv = lax.rsqrt(deg_ref[:, 0] + deg_ref[:, 1] + 1.0)[:, None]
    y = jnp.dot(x_ref[...], w_ref[...], preferred_element_type=jnp.float32)
    y = y * dv
    o_ref[...] = jnp.stack([y[:, :DH // 2], y[:, DH // 2:]], axis=0)
    dis_ref[...] = dv


def _tc_mm1(x, w1, deg):
    return pl.pallas_call(
        _mm1_k,
        grid=(N // _RB,),
        in_specs=[pl.BlockSpec((_RB, DH), lambda i: (i, 0)),
                  pl.BlockSpec((DH, DH), lambda i: (0, 0)),
                  pl.BlockSpec((_RB, NC), lambda i: (i, 0))],
        out_specs=[pl.BlockSpec((NC, _RB, DH // 2), lambda i: (0, i, 0)),
                   pl.BlockSpec((_RB, 1), lambda i: (i, 0))],
        out_shape=[jax.ShapeDtypeStruct((NC, N, DH // 2), jnp.float32),
                   jax.ShapeDtypeStruct((N, 1), jnp.float32)],
    )(x, w1, deg)


def _mm2_k(h_ref, w_ref, dis_ref, o_ref):
    y = (jnp.dot(h_ref[0], w_ref[:DH // 2, :], preferred_element_type=jnp.float32)
         + jnp.dot(h_ref[1], w_ref[DH // 2:, :], preferred_element_type=jnp.float32))
    y = y * dis_ref[...]
    o_ref[...] = jnp.stack([y[:, :DO // 2], y[:, DO // 2:]], axis=0)


def _tc_mm2(h, w2, dis_col):
    return pl.pallas_call(
        _mm2_k,
        grid=(N // _RB,),
        in_specs=[pl.BlockSpec((NC, _RB, DH // 2), lambda i: (0, i, 0)),
                  pl.BlockSpec((DH, DO), lambda i: (0, 0)),
                  pl.BlockSpec((_RB, 1), lambda i: (i, 0))],
        out_specs=pl.BlockSpec((NC, _RB, DO // 2), lambda i: (0, i, 0)),
        out_shape=jax.ShapeDtypeStruct((NC, N, DO // 2), jnp.float32),
    )(h, w2, dis_col)


def _soft_k(z_ref, o_ref):
    z = jnp.concatenate([z_ref[0], z_ref[1]], axis=1)
    z = z - jnp.max(z, axis=1, keepdims=True)
    ez = jnp.exp(z)
    o_ref[...] = ez / jnp.sum(ez, axis=1, keepdims=True)


def _tc_soft(z):
    return pl.pallas_call(
        _soft_k,
        grid=(N // _RB,),
        in_specs=[pl.BlockSpec((NC, _RB, DO // 2), lambda i: (0, i, 0))],
        out_specs=pl.BlockSpec((_RB, DO), lambda i: (i, 0)),
        out_shape=jax.ShapeDtypeStruct((N, DO), jnp.float32),
    )(z)


def kernel(x, edge_index, edge_weight, W1, b1, W2, b2, attention):
    row = edge_index[0].astype(jnp.int32).reshape(E // W, W)
    col = edge_index[1].astype(jnp.int32).reshape(E // W, W)
    ew2 = edge_weight.reshape(E // W, W)
    p0 = jax.nn.softmax(attention, axis=0)[0]
    deg = _sc_deg(col, ew2)
    table1, dis_col = _tc_mm1(x, W1 * p0, deg.T)
    dis = dis_col[:, 0]
    h = _sc_conv(table1, row, col, ew2, b1 * p0, dis,
                 DH // 2, True, 4, False)
    table2 = _tc_mm2(h, W2, dis_col)
    z = _sc_conv(table2, row, col, ew2, b2, dis,
                 DO // 2, False, 8, True)
    return _tc_soft(z)


# speedup vs baseline: 1.2385x; 1.2385x over previous
import functools

import jax
import jax.numpy as jnp
from jax import lax
from jax.experimental import pallas as pl
from jax.experimental.pallas import tpu as pltpu
from jax.experimental.pallas import tpu_sc as plsc

N = 10000
E = 320000
DH = 128
DO = 32
NC = 2
NS = 16
L = 16
NPAD = 10240
RPT = NPAD // NS
W = 125
EPT = E // NS
NWIN = EPT // W
G = 4
BW = 40
WCH = 80
RCH = RPT // WCH

_MESH = plsc.VectorSubcoreMesh(core_axis_name="c", subcore_axis_name="s")


def _splat(ref, i):
    return plsc.load_gather(ref, [jnp.zeros((L,), jnp.int32) + i])


def _zero_rows(ref, nrow, ncol):
    def body(i, _):
        for j in range(ncol // L):
            ref[i, pl.ds(j * L, L)] = jnp.zeros((L,), jnp.float32)
        return 0
    lax.fori_loop(0, nrow, body, 0)


_DEG_WPT = E // NC // NS // W


def _deg_body(coli, ew, deg_out, coli_v, ew_v, zd_v, deg_sh, sem):
    c = lax.axis_index("c")
    s = lax.axis_index("s")
    def zb(i, _):
        zd_v[pl.ds(i * L, L)] = jnp.zeros((L,), jnp.float32)
        return 0
    lax.fori_loop(0, RPT // L, zb, 0)
    pltpu.sync_copy(zd_v, deg_sh.at[pl.ds(s * RPT, RPT)])
    wbase = (c * NS + s) * _DEG_WPT
    pltpu.sync_copy(coli.at[pl.ds(wbase, _DEG_WPT)], coli_v)
    pltpu.sync_copy(ew.at[pl.ds(wbase, _DEG_WPT)], ew_v)
    plsc.subcore_barrier()
    def grp(g, _):
        ds_ = [pltpu.async_copy(ew_v.at[g * 8 + k],
                                deg_sh.at[coli_v.at[g * 8 + k]], sem,
                                add=True)
               for k in range(8)]
        for d in ds_:
            d.wait()
        return 0
    lax.fori_loop(0, _DEG_WPT // 8, grp, 0)
    plsc.subcore_barrier()
    pltpu.sync_copy(deg_sh.at[pl.ds(s * RPT, RPT)],
                    deg_out.at[c, pl.ds(s * RPT, RPT)])


def _sc_deg(coli, ew):
    f = pl.kernel(
        _deg_body,
        out_type=jax.ShapeDtypeStruct((NC, NPAD), jnp.float32),
        mesh=_MESH,
        compiler_params=pltpu.CompilerParams(needs_layout_passes=False, use_tc_tiling_on_sc=False),
        scratch_types=[
            pltpu.VMEM((_DEG_WPT, W), jnp.int32),
            pltpu.VMEM((_DEG_WPT, W), jnp.float32),
            pltpu.VMEM((RPT,), jnp.float32),
            pltpu.VMEM_SHARED((NPAD,), jnp.float32),
            pltpu.SemaphoreType.DMA,
        ],
    )
    return f(coli, ew)


def _conv_body(d2, do_relu, gdep, stage, table, rowi, coli, ew, b, dis_in, out,
               *refs):
    c = lax.axis_index("c")
    s = lax.axis_index("s")
    if stage:
        (rowi_v, coli_v, ew_v, *rbufs, b_v, disc_v, t_sh, acc_sh,
         gsem, ssem) = refs
    else:
        (rowi_v, coli_v, ew_v, *rbufs, b_v, disc_v, acc_sh,
         gsem, ssem) = refs
        t_sh = None
    rb0, rb1 = rbufs[0], rbufs[1]

    _zero_rows(rb0, WCH, d2)
    for k in range(RCH):
        pltpu.sync_copy(rb0.at[pl.ds(0, WCH)], acc_sh.at[pl.ds(s * RPT + k * WCH, WCH)])
    pltpu.sync_copy(b.at[pl.ds(c * d2, d2)], b_v)
    if stage:
        @pl.when(s == 0)
        def _():
            pltpu.sync_copy(table.at[c], t_sh)
    plsc.subcore_barrier()

    def block(blk, _):
        wbase = s * NWIN + blk * BW
        pltpu.sync_copy(rowi.at[pl.ds(wbase, BW)], rowi_v)
        pltpu.sync_copy(coli.at[pl.ds(wbase, BW)], coli_v)
        pltpu.sync_copy(ew.at[pl.ds(wbase, BW)], ew_v)
        def group(g, _):
            if stage:
                gd = [pltpu.async_copy(t_sh.at[rowi_v.at[g * gdep + k]],
                                       rbufs[k], gsem)
                      for k in range(gdep)]
            else:
                gd = [pltpu.async_copy(table.at[c].at[rowi_v.at[g * gdep + k]],
                                       rbufs[k], gsem)
                      for k in range(gdep)]
            sd = []
            for k in range(gdep):
                w = g * gdep + k
                gd[k].wait()
                def scale(i, _):
                    for u in range(5):
                        e = i * 5 + u
                        sp = plsc.load_gather(
                            ew_v, [jnp.zeros((L,), jnp.int32) + w,
                                   jnp.zeros((L,), jnp.int32) + e])
                        for j in range(d2 // L):
                            rbufs[k][e, pl.ds(j * L, L)] = (
                                rbufs[k][e, pl.ds(j * L, L)] * sp)
                    return 0
                lax.fori_loop(0, W // 5, scale, 0)
                sd.append(pltpu.async_copy(rbufs[k], acc_sh.at[coli_v.at[w]],
                                           ssem, add=True))
            for d in sd:
                d.wait()
            return 0
        lax.fori_loop(0, BW // gdep, group, 0)
        return 0
    lax.fori_loop(0, NWIN // BW, block, 0)
    plsc.subcore_barrier()

    def epi(k, _):
        start = s * RPT + k * WCH
        @pl.when(start < N)
        def _():
            pltpu.sync_copy(acc_sh.at[pl.ds(start, WCH)], rb0.at[pl.ds(0, WCH)])
            if stage:
                pltpu.sync_copy(t_sh.at[pl.ds(start, WCH)], rb1.at[pl.ds(0, WCH)])
            else:
                pltpu.sync_copy(table.at[c, pl.ds(start, WCH)], rb1.at[pl.ds(0, WCH)])
            pltpu.sync_copy(dis_in.at[pl.ds(start, WCH)], disc_v)
            def erow(r, _):
                dsp = _splat(disc_v, r)
                for j in range(d2 // L):
                    a = rb0[r, pl.ds(j * L, L)]
                    t = rb1[r, pl.ds(j * L, L)]
                    bb = b_v[pl.ds(j * L, L)]
                    v = dsp * (a + t) + bb
                    if do_relu:
                        v = jnp.maximum(v, 0.0)
                    rb0[r, pl.ds(j * L, L)] = v
                return 0
            lax.fori_loop(0, WCH, erow, 0)
            pltpu.sync_copy(rb0.at[pl.ds(0, WCH)], out.at[c, pl.ds(start, WCH)])
        return 0
    lax.fori_loop(0, RCH, epi, 0)


def _sc_conv(table, rowi, coli, ew, b, dis, d2, do_relu, gdep, stage):
    scratch = [
        pltpu.VMEM((BW, W), jnp.int32),
        pltpu.VMEM((BW, W), jnp.int32),
        pltpu.VMEM((BW, W), jnp.float32),
    ]
    scratch += [pltpu.VMEM((W, d2), jnp.float32) for _ in range(gdep)]
    scratch += [
        pltpu.VMEM((d2,), jnp.float32),
        pltpu.VMEM((WCH,), jnp.float32),
    ]
    if stage:
        scratch.append(pltpu.VMEM_SHARED((N, d2), jnp.float32))
    scratch += [
        pltpu.VMEM_SHARED((NPAD, d2), jnp.float32),
        pltpu.SemaphoreType.DMA,
        pltpu.SemaphoreType.DMA,
    ]
    f = pl.kernel(
        functools.partial(_conv_body, d2, do_relu, gdep, stage),
        out_type=jax.ShapeDtypeStruct((NC, N, d2), jnp.float32),
        mesh=_MESH,
        compiler_params=pltpu.CompilerParams(needs_layout_passes=False, use_tc_tiling_on_sc=False),
        scratch_types=scratch,
    )
    return f(table, rowi, coli, ew, b, dis)


_WPT2 = E // NC // NS // W


def _acc2_body(table, rowi, coli, ew, acc_out,
               rowi_v, coli_v, ew_v, rb0, rb1, rb2, rb3,
               acc_sh, gsem, ssem):
    c = lax.axis_index("c")
    s = lax.axis_index("s")
    rbufs = (rb0, rb1, rb2, rb3)

    _zero_rows(rb0, WCH, DO)
    for k in range(RCH):
        pltpu.sync_copy(rb0.at[pl.ds(0, WCH)],
                        acc_sh.at[pl.ds(s * RPT + k * WCH, WCH)])
    plsc.subcore_barrier()

    def block(blk, _):
        wbase = (c * NS + s) * _WPT2 + blk * BW
        pltpu.sync_copy(rowi.at[pl.ds(wbase, BW)], rowi_v)
        pltpu.sync_copy(coli.at[pl.ds(wbase, BW)], coli_v)
        pltpu.sync_copy(ew.at[pl.ds(wbase, BW)], ew_v)
        def group(g, _):
            gd = [pltpu.async_copy(table.at[rowi_v.at[g * G + k]],
                                   rbufs[k], gsem)
                  for k in range(G)]
            sd = []
            for k in range(G):
                w = g * G + k
                gd[k].wait()
                def scale(i, _):
                    for u in range(5):
                        e = i * 5 + u
                        sp = plsc.load_gather(
                            ew_v, [jnp.zeros((L,), jnp.int32) + w,
                                   jnp.zeros((L,), jnp.int32) + e])
                        for j in range(DO // L):
                            rbufs[k][e, pl.ds(j * L, L)] = (
                                rbufs[k][e, pl.ds(j * L, L)] * sp)
                    return 0
                lax.fori_loop(0, W // 5, scale, 0)
                sd.append(pltpu.async_copy(rbufs[k], acc_sh.at[coli_v.at[w]],
                                           ssem, add=True))
            for d in sd:
                d.wait()
            return 0
        lax.fori_loop(0, BW // G, group, 0)
        return 0
    lax.fori_loop(0, _WPT2 // BW, block, 0)
    plsc.subcore_barrier()
    pltpu.sync_copy(acc_sh.at[pl.ds(s * RPT, RPT)],
                    acc_out.at[c, pl.ds(s * RPT, RPT)])


def _sc_acc2(table, rowi, coli, ew):
    f = pl.kernel(
        _acc2_body,
        out_type=jax.ShapeDtypeStruct((NC, NPAD, DO), jnp.float32),
        mesh=_MESH,
        compiler_params=pltpu.CompilerParams(needs_layout_passes=False, use_tc_tiling_on_sc=False),
        scratch_types=[
            pltpu.VMEM((BW, W), jnp.int32),
            pltpu.VMEM((BW, W), jnp.int32),
            pltpu.VMEM((BW, W), jnp.float32),
            pltpu.VMEM((W, DO), jnp.float32),
            pltpu.VMEM((W, DO), jnp.float32),
            pltpu.VMEM((W, DO), jnp.float32),
            pltpu.VMEM((W, DO), jnp.float32),
            pltpu.VMEM_SHARED((NPAD, DO), jnp.float32),
            pltpu.SemaphoreType.DMA,
            pltpu.SemaphoreType.DMA,
        ],
    )
    return f(table, rowi, coli, ew)
_RB = 1000


def _mm1_k(x_ref, w_ref, deg_ref, o_ref, dis_ref):
    dv = lax.rsqrt(deg_ref[:, 0] + deg_ref[:, 1] + 1.0)[:, None]
    y = jnp.dot(x_ref[...], w_ref[...], preferred_element_type=jnp.float32)
    y = y * dv
    o_ref[...] = jnp.stack([y[:, :DH // 2], y[:, DH // 2:]], axis=0)
    dis_ref[...] = dv


def _tc_mm1(x, w1, deg):
    return pl.pallas_call(
        _mm1_k,
        grid=(N // _RB,),
        in_specs=[pl.BlockSpec((_RB, DH), lambda i: (i, 0)),
                  pl.BlockSpec((DH, DH), lambda i: (0, 0)),
                  pl.BlockSpec((_RB, NC), lambda i: (i, 0))],
        out_specs=[pl.BlockSpec((NC, _RB, DH // 2), lambda i: (0, i, 0)),
                   pl.BlockSpec((_RB, 1), lambda i: (i, 0))],
        out_shape=[jax.ShapeDtypeStruct((NC, N, DH // 2), jnp.float32),
                   jax.ShapeDtypeStruct((N, 1), jnp.float32)],
    )(x, w1, deg)


def _mm2_k(h_ref, w_ref, dis_ref, o_ref):
    y = (jnp.dot(h_ref[0], w_ref[:DH // 2, :], preferred_element_type=jnp.float32)
         + jnp.dot(h_ref[1], w_ref[DH // 2:, :], preferred_element_type=jnp.float32))
    o_ref[...] = y * dis_ref[...]


def _tc_mm2(h, w2, dis_col):
    return pl.pallas_call(
        _mm2_k,
        grid=(N // _RB,),
        in_specs=[pl.BlockSpec((NC, _RB, DH // 2), lambda i: (0, i, 0)),
                  pl.BlockSpec((DH, DO), lambda i: (0, 0)),
                  pl.BlockSpec((_RB, 1), lambda i: (i, 0))],
        out_specs=pl.BlockSpec((_RB, DO), lambda i: (i, 0)),
        out_shape=jax.ShapeDtypeStruct((N, DO), jnp.float32),
    )(h, w2, dis_col)


def _soft_k(acc_ref, t2_ref, dis_ref, b2_ref, o_ref):
    z = dis_ref[...] * (acc_ref[0] + acc_ref[1] + t2_ref[...]) + b2_ref[...]
    z = z - jnp.max(z, axis=1, keepdims=True)
    ez = jnp.exp(z)
    o_ref[...] = ez / jnp.sum(ez, axis=1, keepdims=True)


def _tc_soft(acc, t2, dis_col, b2):
    return pl.pallas_call(
        _soft_k,
        grid=(N // _RB,),
        in_specs=[pl.BlockSpec((NC, _RB, DO), lambda i: (0, i, 0)),
                  pl.BlockSpec((_RB, DO), lambda i: (i, 0)),
                  pl.BlockSpec((_RB, 1), lambda i: (i, 0)),
                  pl.BlockSpec((1, DO), lambda i: (0, 0))],
        out_specs=pl.BlockSpec((_RB, DO), lambda i: (i, 0)),
        out_shape=jax.ShapeDtypeStruct((N, DO), jnp.float32),
    )(acc, t2, dis_col, b2)


def kernel(x, edge_index, edge_weight, W1, b1, W2, b2, attention):
    row = edge_index[0].astype(jnp.int32).reshape(E // W, W)
    col = edge_index[1].astype(jnp.int32).reshape(E // W, W)
    ew2 = edge_weight.reshape(E // W, W)
    p0 = jax.nn.softmax(attention, axis=0)[0]
    deg = _sc_deg(col, ew2)
    table1, dis_col = _tc_mm1(x, W1 * p0, deg.T)
    dis = dis_col[:, 0]
    h = _sc_conv(table1, row, col, ew2, b1 * p0, dis,
                 DH // 2, True, 4, False)
    table2 = _tc_mm2(h, W2, dis_col)
    acc2 = _sc_acc2(table2, row, col, ew2)
    return _tc_soft(acc2, table2, dis_col, b2[None, :])


# speedup vs baseline: 1.6188x; 1.3071x over previous
import functools

import jax
import jax.numpy as jnp
from jax import lax
from jax.experimental import pallas as pl
from jax.experimental.pallas import tpu as pltpu
from jax.experimental.pallas import tpu_sc as plsc

N = 10000
E = 320000
DH = 128
DO = 32
NC = 2
NS = 16
L = 16
NPAD = 10240
RPT = NPAD // NS
W = 125
EPT = E // NS
NWIN = EPT // W
G = 4
BW = 40
WCH = 80
RCH = RPT // WCH

_MESH = plsc.VectorSubcoreMesh(core_axis_name="c", subcore_axis_name="s")


def _splat(ref, i):
    return plsc.load_gather(ref, [jnp.zeros((L,), jnp.int32) + i])


def _zero_rows(ref, nrow, ncol):
    def body(i, _):
        for j in range(ncol // L):
            ref[i, pl.ds(j * L, L)] = jnp.zeros((L,), jnp.float32)
        return 0
    lax.fori_loop(0, nrow, body, 0)


_DEG_WPT = E // NC // NS // W


def _deg_body(coli, ew, deg_out, coli_v, ew_v, zd_v, deg_sh, sem):
    c = lax.axis_index("c")
    s = lax.axis_index("s")
    def zb(i, _):
        zd_v[pl.ds(i * L, L)] = jnp.zeros((L,), jnp.float32)
        return 0
    lax.fori_loop(0, RPT // L, zb, 0)
    pltpu.sync_copy(zd_v, deg_sh.at[pl.ds(s * RPT, RPT)])
    wbase = (c * NS + s) * _DEG_WPT
    pltpu.sync_copy(coli.at[pl.ds(wbase, _DEG_WPT)], coli_v)
    pltpu.sync_copy(ew.at[pl.ds(wbase, _DEG_WPT)], ew_v)
    plsc.subcore_barrier()
    def grp(g, _):
        ds_ = [pltpu.async_copy(ew_v.at[g * 8 + k],
                                deg_sh.at[coli_v.at[g * 8 + k]], sem,
                                add=True)
               for k in range(8)]
        for d in ds_:
            d.wait()
        return 0
    lax.fori_loop(0, _DEG_WPT // 8, grp, 0)
    plsc.subcore_barrier()
    pltpu.sync_copy(deg_sh.at[pl.ds(s * RPT, RPT)],
                    deg_out.at[c, pl.ds(s * RPT, RPT)])


def _sc_deg(coli, ew):
    f = pl.kernel(
        _deg_body,
        out_type=jax.ShapeDtypeStruct((NC, NPAD), jnp.float32),
        mesh=_MESH,
        compiler_params=pltpu.CompilerParams(needs_layout_passes=False, use_tc_tiling_on_sc=False),
        scratch_types=[
            pltpu.VMEM((_DEG_WPT, W), jnp.int32),
            pltpu.VMEM((_DEG_WPT, W), jnp.float32),
            pltpu.VMEM((RPT,), jnp.float32),
            pltpu.VMEM_SHARED((NPAD,), jnp.float32),
            pltpu.SemaphoreType.DMA,
        ],
    )
    return f(coli, ew)


def _conv_body(d2, do_relu, gdep, stage, table, rowi, coli, ew, b, dis_in, out,
               *refs):
    c = lax.axis_index("c")
    s = lax.axis_index("s")
    if stage:
        (rowi_v, coli_v, ew_v, *rbufs, b_v, disc_v, t_sh, acc_sh,
         gsem, ssem) = refs
    else:
        (rowi_v, coli_v, ew_v, *rbufs, b_v, disc_v, acc_sh,
         gsem, ssem) = refs
        t_sh = None
    rb0, rb1 = rbufs[0], rbufs[1]

    _zero_rows(rb0, WCH, d2)
    for k in range(RCH):
        pltpu.sync_copy(rb0.at[pl.ds(0, WCH)], acc_sh.at[pl.ds(s * RPT + k * WCH, WCH)])
    pltpu.sync_copy(b.at[pl.ds(c * d2, d2)], b_v)
    if stage:
        @pl.when(s == 0)
        def _():
            pltpu.sync_copy(table.at[c], t_sh)
    plsc.subcore_barrier()

    def block(blk, _):
        wbase = s * NWIN + blk * BW
        pltpu.sync_copy(rowi.at[pl.ds(wbase, BW)], rowi_v)
        pltpu.sync_copy(coli.at[pl.ds(wbase, BW)], coli_v)
        pltpu.sync_copy(ew.at[pl.ds(wbase, BW)], ew_v)
        def group(g, _):
            if stage:
                gd = [pltpu.async_copy(t_sh.at[rowi_v.at[g * gdep + k]],
                                       rbufs[k], gsem)
                      for k in range(gdep)]
            else:
                gd = [pltpu.async_copy(table.at[c].at[rowi_v.at[g * gdep + k]],
                                       rbufs[k], gsem)
                      for k in range(gdep)]
            sd = []
            for k in range(gdep):
                w = g * gdep + k
                gd[k].wait()
                @plsc.parallel_loop(0, W // 5)
                def scale(i):
                    for u in range(5):
                        e = i * 5 + u
                        sp = plsc.load_gather(
                            ew_v, [jnp.zeros((L,), jnp.int32) + w,
                                   jnp.zeros((L,), jnp.int32) + e])
                        for j in range(d2 // L):
                            rbufs[k][e, pl.ds(j * L, L)] = (
                                rbufs[k][e, pl.ds(j * L, L)] * sp)
                sd.append(pltpu.async_copy(rbufs[k], acc_sh.at[coli_v.at[w]],
                                           ssem, add=True))
            for d in sd:
                d.wait()
            return 0
        lax.fori_loop(0, BW // gdep, group, 0)
        return 0
    lax.fori_loop(0, NWIN // BW, block, 0)
    plsc.subcore_barrier()

    def epi(k, _):
        start = s * RPT + k * WCH
        @pl.when(start < N)
        def _():
            pltpu.sync_copy(acc_sh.at[pl.ds(start, WCH)], rb0.at[pl.ds(0, WCH)])
            if stage:
                pltpu.sync_copy(t_sh.at[pl.ds(start, WCH)], rb1.at[pl.ds(0, WCH)])
            else:
                pltpu.sync_copy(table.at[c, pl.ds(start, WCH)], rb1.at[pl.ds(0, WCH)])
            pltpu.sync_copy(dis_in.at[pl.ds(start, WCH)], disc_v)
            def erow(r, _):
                dsp = _splat(disc_v, r)
                for j in range(d2 // L):
                    a = rb0[r, pl.ds(j * L, L)]
                    t = rb1[r, pl.ds(j * L, L)]
                    bb = b_v[pl.ds(j * L, L)]
                    v = dsp * (a + t) + bb
                    if do_relu:
                        v = jnp.maximum(v, 0.0)
                    rb0[r, pl.ds(j * L, L)] = v
                return 0
            lax.fori_loop(0, WCH, erow, 0)
            pltpu.sync_copy(rb0.at[pl.ds(0, WCH)], out.at[c, pl.ds(start, WCH)])
        return 0
    lax.fori_loop(0, RCH, epi, 0)


def _sc_conv(table, rowi, coli, ew, b, dis, d2, do_relu, gdep, stage):
    scratch = [
        pltpu.VMEM((BW, W), jnp.int32),
        pltpu.VMEM((BW, W), jnp.int32),
        pltpu.VMEM((BW, W), jnp.float32),
    ]
    scratch += [pltpu.VMEM((W, d2), jnp.float32) for _ in range(gdep)]
    scratch += [
        pltpu.VMEM((d2,), jnp.float32),
        pltpu.VMEM((WCH,), jnp.float32),
    ]
    if stage:
        scratch.append(pltpu.VMEM_SHARED((N, d2), jnp.float32))
    scratch += [
        pltpu.VMEM_SHARED((NPAD, d2), jnp.float32),
        pltpu.SemaphoreType.DMA,
        pltpu.SemaphoreType.DMA,
    ]
    f = pl.kernel(
        functools.partial(_conv_body, d2, do_relu, gdep, stage),
        out_type=jax.ShapeDtypeStruct((NC, N, d2), jnp.float32),
        mesh=_MESH,
        compiler_params=pltpu.CompilerParams(needs_layout_passes=False, use_tc_tiling_on_sc=False),
        scratch_types=scratch,
    )
    return f(table, rowi, coli, ew, b, dis)


_WPT2 = E // NC // NS // W


def _acc2_body(table, rowi, coli, ew, acc_out,
               rowi_v, coli_v, ew_v, rb0, rb1, rb2, rb3,
               acc_sh, gsem, ssem):
    c = lax.axis_index("c")
    s = lax.axis_index("s")
    rbufs = (rb0, rb1, rb2, rb3)

    _zero_rows(rb0, WCH, DO)
    for k in range(RCH):
        pltpu.sync_copy(rb0.at[pl.ds(0, WCH)],
                        acc_sh.at[pl.ds(s * RPT + k * WCH, WCH)])
    plsc.subcore_barrier()

    def block(blk, _):
        wbase = (c * NS + s) * _WPT2 + blk * BW
        pltpu.sync_copy(rowi.at[pl.ds(wbase, BW)], rowi_v)
        pltpu.sync_copy(coli.at[pl.ds(wbase, BW)], coli_v)
        pltpu.sync_copy(ew.at[pl.ds(wbase, BW)], ew_v)
        def group(g, _):
            gd = [pltpu.async_copy(table.at[rowi_v.at[g * G + k]],
                                   rbufs[k], gsem)
                  for k in range(G)]
            sd = []
            for k in range(G):
                w = g * G + k
                gd[k].wait()
                @plsc.parallel_loop(0, W // 5)
                def scale(i):
                    for u in range(5):
                        e = i * 5 + u
                        sp = plsc.load_gather(
                            ew_v, [jnp.zeros((L,), jnp.int32) + w,
                                   jnp.zeros((L,), jnp.int32) + e])
                        for j in range(DO // L):
                            rbufs[k][e, pl.ds(j * L, L)] = (
                                rbufs[k][e, pl.ds(j * L, L)] * sp)
                sd.append(pltpu.async_copy(rbufs[k], acc_sh.at[coli_v.at[w]],
                                           ssem, add=True))
            for d in sd:
                d.wait()
            return 0
        lax.fori_loop(0, BW // G, group, 0)
        return 0
    lax.fori_loop(0, _WPT2 // BW, block, 0)
    plsc.subcore_barrier()
    pltpu.sync_copy(acc_sh.at[pl.ds(s * RPT, RPT)],
                    acc_out.at[c, pl.ds(s * RPT, RPT)])


def _sc_acc2(table, rowi, coli, ew):
    f = pl.kernel(
        _acc2_body,
        out_type=jax.ShapeDtypeStruct((NC, NPAD, DO), jnp.float32),
        mesh=_MESH,
        compiler_params=pltpu.CompilerParams(needs_layout_passes=False, use_tc_tiling_on_sc=False),
        scratch_types=[
            pltpu.VMEM((BW, W), jnp.int32),
            pltpu.VMEM((BW, W), jnp.int32),
            pltpu.VMEM((BW, W), jnp.float32),
            pltpu.VMEM((W, DO), jnp.float32),
            pltpu.VMEM((W, DO), jnp.float32),
            pltpu.VMEM((W, DO), jnp.float32),
            pltpu.VMEM((W, DO), jnp.float32),
            pltpu.VMEM_SHARED((NPAD, DO), jnp.float32),
            pltpu.SemaphoreType.DMA,
            pltpu.SemaphoreType.DMA,
        ],
    )
    return f(table, rowi, coli, ew)
_RB = 1000


def _mm1_k(x_ref, w_ref, deg_ref, o_ref, dis_ref):
    dv = lax.rsqrt(deg_ref[:, 0] + deg_ref[:, 1] + 1.0)[:, None]
    y = jnp.dot(x_ref[...], w_ref[...], preferred_element_type=jnp.float32)
    y = y * dv
    o_ref[...] = jnp.stack([y[:, :DH // 2], y[:, DH // 2:]], axis=0)
    dis_ref[...] = dv


def _tc_mm1(x, w1, deg):
    return pl.pallas_call(
        _mm1_k,
        grid=(N // _RB,),
        in_specs=[pl.BlockSpec((_RB, DH), lambda i: (i, 0)),
                  pl.BlockSpec((DH, DH), lambda i: (0, 0)),
                  pl.BlockSpec((_RB, NC), lambda i: (i, 0))],
        out_specs=[pl.BlockSpec((NC, _RB, DH // 2), lambda i: (0, i, 0)),
                   pl.BlockSpec((_RB, 1), lambda i: (i, 0))],
        out_shape=[jax.ShapeDtypeStruct((NC, N, DH // 2), jnp.float32),
                   jax.ShapeDtypeStruct((N, 1), jnp.float32)],
    )(x, w1, deg)


def _mm2_k(h_ref, w_ref, dis_ref, o_ref):
    y = (jnp.dot(h_ref[0], w_ref[:DH // 2, :], preferred_element_type=jnp.float32)
         + jnp.dot(h_ref[1], w_ref[DH // 2:, :], preferred_element_type=jnp.float32))
    o_ref[...] = y * dis_ref[...]


def _tc_mm2(h, w2, dis_col):
    return pl.pallas_call(
        _mm2_k,
        grid=(N // _RB,),
        in_specs=[pl.BlockSpec((NC, _RB, DH // 2), lambda i: (0, i, 0)),
                  pl.BlockSpec((DH, DO), lambda i: (0, 0)),
                  pl.BlockSpec((_RB, 1), lambda i: (i, 0))],
        out_specs=pl.BlockSpec((_RB, DO), lambda i: (i, 0)),
        out_shape=jax.ShapeDtypeStruct((N, DO), jnp.float32),
    )(h, w2, dis_col)


def _soft_k(acc_ref, t2_ref, dis_ref, b2_ref, o_ref):
    z = dis_ref[...] * (acc_ref[0] + acc_ref[1] + t2_ref[...]) + b2_ref[...]
    z = z - jnp.max(z, axis=1, keepdims=True)
    ez = jnp.exp(z)
    o_ref[...] = ez / jnp.sum(ez, axis=1, keepdims=True)


def _tc_soft(acc, t2, dis_col, b2):
    return pl.pallas_call(
        _soft_k,
        grid=(N // _RB,),
        in_specs=[pl.BlockSpec((NC, _RB, DO), lambda i: (0, i, 0)),
                  pl.BlockSpec((_RB, DO), lambda i: (i, 0)),
                  pl.BlockSpec((_RB, 1), lambda i: (i, 0)),
                  pl.BlockSpec((1, DO), lambda i: (0, 0))],
        out_specs=pl.BlockSpec((_RB, DO), lambda i: (i, 0)),
        out_shape=jax.ShapeDtypeStruct((N, DO), jnp.float32),
    )(acc, t2, dis_col, b2)


def kernel(x, edge_index, edge_weight, W1, b1, W2, b2, attention):
    row = edge_index[0].astype(jnp.int32).reshape(E // W, W)
    col = edge_index[1].astype(jnp.int32).reshape(E // W, W)
    ew2 = edge_weight.reshape(E // W, W)
    p0 = jax.nn.softmax(attention, axis=0)[0]
    deg = _sc_deg(col, ew2)
    table1, dis_col = _tc_mm1(x, W1 * p0, deg.T)
    dis = dis_col[:, 0]
    h = _sc_conv(table1, row, col, ew2, b1 * p0, dis,
                 DH // 2, True, 8, False)
    table2 = _tc_mm2(h, W2, dis_col)
    acc2 = _sc_acc2(table2, row, col, ew2)
    return _tc_soft(acc2, table2, dis_col, b2[None, :])


# speedup vs baseline: 1.6290x; 1.0063x over previous
import functools

import jax
import jax.numpy as jnp
from jax import lax
from jax.experimental import pallas as pl
from jax.experimental.pallas import tpu as pltpu
from jax.experimental.pallas import tpu_sc as plsc

N = 10000
E = 320000
DH = 128
DO = 32
NC = 2
NS = 16
L = 16
NPAD = 10240
RPT = NPAD // NS
W = 125
EPT = E // NS
NWIN = EPT // W
G = 4
BW = 40
WCH = 80
RCH = RPT // WCH

_MESH = plsc.VectorSubcoreMesh(core_axis_name="c", subcore_axis_name="s")


def _splat(ref, i):
    return plsc.load_gather(ref, [jnp.zeros((L,), jnp.int32) + i])


def _zero_rows(ref, nrow, ncol):
    def body(i, _):
        for j in range(ncol // L):
            ref[i, pl.ds(j * L, L)] = jnp.zeros((L,), jnp.float32)
        return 0
    lax.fori_loop(0, nrow, body, 0)


_DEG_WPT = E // NC // NS // W


def _deg_body(coli, ew, deg_out, coli_v, ew_v, zd_v, deg_sh, sem):
    c = lax.axis_index("c")
    s = lax.axis_index("s")
    def zb(i, _):
        zd_v[pl.ds(i * L, L)] = jnp.zeros((L,), jnp.float32)
        return 0
    lax.fori_loop(0, RPT // L, zb, 0)
    pltpu.sync_copy(zd_v, deg_sh.at[pl.ds(s * RPT, RPT)])
    wbase = (c * NS + s) * _DEG_WPT
    pltpu.sync_copy(coli.at[pl.ds(wbase, _DEG_WPT)], coli_v)
    pltpu.sync_copy(ew.at[pl.ds(wbase, _DEG_WPT)], ew_v)
    plsc.subcore_barrier()
    def grp(g, _):
        ds_ = [pltpu.async_copy(ew_v.at[g * 8 + k],
                                deg_sh.at[coli_v.at[g * 8 + k]], sem,
                                add=True)
               for k in range(8)]
        for d in ds_:
            d.wait()
        return 0
    lax.fori_loop(0, _DEG_WPT // 8, grp, 0)
    plsc.subcore_barrier()
    pltpu.sync_copy(deg_sh.at[pl.ds(s * RPT, RPT)],
                    deg_out.at[c, pl.ds(s * RPT, RPT)])


def _sc_deg(coli, ew):
    f = pl.kernel(
        _deg_body,
        out_type=jax.ShapeDtypeStruct((NC, NPAD), jnp.float32),
        mesh=_MESH,
        compiler_params=pltpu.CompilerParams(needs_layout_passes=False, use_tc_tiling_on_sc=False),
        scratch_types=[
            pltpu.VMEM((_DEG_WPT, W), jnp.int32),
            pltpu.VMEM((_DEG_WPT, W), jnp.float32),
            pltpu.VMEM((RPT,), jnp.float32),
            pltpu.VMEM_SHARED((NPAD,), jnp.float32),
            pltpu.SemaphoreType.DMA,
        ],
    )
    return f(coli, ew)


def _conv_body(d2, do_relu, gdep, stage, table, rowi, coli, ew, b, dis_in, out,
               *refs):
    c = lax.axis_index("c")
    s = lax.axis_index("s")
    if stage:
        (rowi_v, coli_v, ew_v, *rbufs, b_v, disc_v, t_sh, acc_sh,
         gsem, ssem) = refs
    else:
        (rowi_v, coli_v, ew_v, *rbufs, b_v, disc_v, acc_sh,
         gsem, ssem) = refs
        t_sh = None
    rb0, rb1 = rbufs[0], rbufs[1]

    _zero_rows(rb0, WCH, d2)
    for k in range(RCH):
        pltpu.sync_copy(rb0.at[pl.ds(0, WCH)], acc_sh.at[pl.ds(s * RPT + k * WCH, WCH)])
    pltpu.sync_copy(b.at[pl.ds(c * d2, d2)], b_v)
    if stage:
        @pl.when(s == 0)
        def _():
            pltpu.sync_copy(table.at[c], t_sh)
    plsc.subcore_barrier()

    def block(blk, _):
        wbase = s * NWIN + blk * BW
        pltpu.sync_copy(rowi.at[pl.ds(wbase, BW)], rowi_v)
        pltpu.sync_copy(coli.at[pl.ds(wbase, BW)], coli_v)
        pltpu.sync_copy(ew.at[pl.ds(wbase, BW)], ew_v)
        def group(g, _):
            if stage:
                gd = [pltpu.async_copy(t_sh.at[rowi_v.at[g * gdep + k]],
                                       rbufs[k], gsem)
                      for k in range(gdep)]
            else:
                gd = [pltpu.async_copy(table.at[c].at[rowi_v.at[g * gdep + k]],
                                       rbufs[k], gsem)
                      for k in range(gdep)]
            sd = []
            for k in range(gdep):
                w = g * gdep + k
                gd[k].wait()
                @plsc.parallel_loop(0, W, unroll=5)
                def scale(e):
                    sp = plsc.load_gather(
                        ew_v, [jnp.zeros((L,), jnp.int32) + w,
                               jnp.zeros((L,), jnp.int32) + e])
                    for j in range(d2 // L):
                        rbufs[k][e, pl.ds(j * L, L)] = (
                            rbufs[k][e, pl.ds(j * L, L)] * sp)
                sd.append(pltpu.async_copy(rbufs[k], acc_sh.at[coli_v.at[w]],
                                           ssem, add=True))
            for d in sd:
                d.wait()
            return 0
        lax.fori_loop(0, BW // gdep, group, 0)
        return 0
    lax.fori_loop(0, NWIN // BW, block, 0)
    plsc.subcore_barrier()

    def epi(k, _):
        start = s * RPT + k * WCH
        @pl.when(start < N)
        def _():
            pltpu.sync_copy(acc_sh.at[pl.ds(start, WCH)], rb0.at[pl.ds(0, WCH)])
            if stage:
                pltpu.sync_copy(t_sh.at[pl.ds(start, WCH)], rb1.at[pl.ds(0, WCH)])
            else:
                pltpu.sync_copy(table.at[c, pl.ds(start, WCH)], rb1.at[pl.ds(0, WCH)])
            pltpu.sync_copy(dis_in.at[pl.ds(start, WCH)], disc_v)
            def erow(r, _):
                dsp = _splat(disc_v, r)
                for j in range(d2 // L):
                    a = rb0[r, pl.ds(j * L, L)]
                    t = rb1[r, pl.ds(j * L, L)]
                    bb = b_v[pl.ds(j * L, L)]
                    v = dsp * (a + t) + bb
                    if do_relu:
                        v = jnp.maximum(v, 0.0)
                    rb0[r, pl.ds(j * L, L)] = v
                return 0
            lax.fori_loop(0, WCH, erow, 0)
            pltpu.sync_copy(rb0.at[pl.ds(0, WCH)], out.at[c, pl.ds(start, WCH)])
        return 0
    lax.fori_loop(0, RCH, epi, 0)


def _sc_conv(table, rowi, coli, ew, b, dis, d2, do_relu, gdep, stage):
    scratch = [
        pltpu.VMEM((BW, W), jnp.int32),
        pltpu.VMEM((BW, W), jnp.int32),
        pltpu.VMEM((BW, W), jnp.float32),
    ]
    scratch += [pltpu.VMEM((W, d2), jnp.float32) for _ in range(gdep)]
    scratch += [
        pltpu.VMEM((d2,), jnp.float32),
        pltpu.VMEM((WCH,), jnp.float32),
    ]
    if stage:
        scratch.append(pltpu.VMEM_SHARED((N, d2), jnp.float32))
    scratch += [
        pltpu.VMEM_SHARED((NPAD, d2), jnp.float32),
        pltpu.SemaphoreType.DMA,
        pltpu.SemaphoreType.DMA,
    ]
    f = pl.kernel(
        functools.partial(_conv_body, d2, do_relu, gdep, stage),
        out_type=jax.ShapeDtypeStruct((NC, N, d2), jnp.float32),
        mesh=_MESH,
        compiler_params=pltpu.CompilerParams(needs_layout_passes=False, use_tc_tiling_on_sc=False),
        scratch_types=scratch,
    )
    return f(table, rowi, coli, ew, b, dis)


_WPT2 = E // NC // NS // W


def _acc2_body(table, rowi, coli, ew, acc_out,
               rowi_v, coli_v, ew_v, rb0, rb1, rb2, rb3,
               acc_sh, gsem, ssem):
    c = lax.axis_index("c")
    s = lax.axis_index("s")
    rbufs = (rb0, rb1, rb2, rb3)

    _zero_rows(rb0, WCH, DO)
    for k in range(RCH):
        pltpu.sync_copy(rb0.at[pl.ds(0, WCH)],
                        acc_sh.at[pl.ds(s * RPT + k * WCH, WCH)])
    plsc.subcore_barrier()

    def block(blk, _):
        wbase = (c * NS + s) * _WPT2 + blk * BW
        pltpu.sync_copy(rowi.at[pl.ds(wbase, BW)], rowi_v)
        pltpu.sync_copy(coli.at[pl.ds(wbase, BW)], coli_v)
        pltpu.sync_copy(ew.at[pl.ds(wbase, BW)], ew_v)
        def group(g, _):
            gd = [pltpu.async_copy(table.at[rowi_v.at[g * G + k]],
                                   rbufs[k], gsem)
                  for k in range(G)]
            sd = []
            for k in range(G):
                w = g * G + k
                gd[k].wait()
                @plsc.parallel_loop(0, W, unroll=5)
                def scale(e):
                    sp = plsc.load_gather(
                        ew_v, [jnp.zeros((L,), jnp.int32) + w,
                               jnp.zeros((L,), jnp.int32) + e])
                    for j in range(DO // L):
                        rbufs[k][e, pl.ds(j * L, L)] = (
                            rbufs[k][e, pl.ds(j * L, L)] * sp)
                sd.append(pltpu.async_copy(rbufs[k], acc_sh.at[coli_v.at[w]],
                                           ssem, add=True))
            for d in sd:
                d.wait()
            return 0
        lax.fori_loop(0, BW // G, group, 0)
        return 0
    lax.fori_loop(0, _WPT2 // BW, block, 0)
    plsc.subcore_barrier()
    pltpu.sync_copy(acc_sh.at[pl.ds(s * RPT, RPT)],
                    acc_out.at[c, pl.ds(s * RPT, RPT)])


def _sc_acc2(table, rowi, coli, ew):
    f = pl.kernel(
        _acc2_body,
        out_type=jax.ShapeDtypeStruct((NC, NPAD, DO), jnp.float32),
        mesh=_MESH,
        compiler_params=pltpu.CompilerParams(needs_layout_passes=False, use_tc_tiling_on_sc=False),
        scratch_types=[
            pltpu.VMEM((BW, W), jnp.int32),
            pltpu.VMEM((BW, W), jnp.int32),
            pltpu.VMEM((BW, W), jnp.float32),
            pltpu.VMEM((W, DO), jnp.float32),
            pltpu.VMEM((W, DO), jnp.float32),
            pltpu.VMEM((W, DO), jnp.float32),
            pltpu.VMEM((W, DO), jnp.float32),
            pltpu.VMEM_SHARED((NPAD, DO), jnp.float32),
            pltpu.SemaphoreType.DMA,
            pltpu.SemaphoreType.DMA,
        ],
    )
    return f(table, rowi, coli, ew)
_RB = 1000


def _mm1_k(x_ref, w_ref, deg_ref, o_ref, dis_ref):
    dv = lax.rsqrt(deg_ref[:, 0] + deg_ref[:, 1] + 1.0)[:, None]
    y = jnp.dot(x_ref[...], w_ref[...], preferred_element_type=jnp.float32)
    y = y * dv
    o_ref[...] = jnp.stack([y[:, :DH // 2], y[:, DH // 2:]], axis=0)
    dis_ref[...] = dv


def _tc_mm1(x, w1, deg):
    return pl.pallas_call(
        _mm1_k,
        grid=(N // _RB,),
        in_specs=[pl.BlockSpec((_RB, DH), lambda i: (i, 0)),
                  pl.BlockSpec((DH, DH), lambda i: (0, 0)),
                  pl.BlockSpec((_RB, NC), lambda i: (i, 0))],
        out_specs=[pl.BlockSpec((NC, _RB, DH // 2), lambda i: (0, i, 0)),
                   pl.BlockSpec((_RB, 1), lambda i: (i, 0))],
        out_shape=[jax.ShapeDtypeStruct((NC, N, DH // 2), jnp.float32),
                   jax.ShapeDtypeStruct((N, 1), jnp.float32)],
    )(x, w1, deg)


def _mm2_k(h_ref, w_ref, dis_ref, o_ref):
    y = (jnp.dot(h_ref[0], w_ref[:DH // 2, :], preferred_element_type=jnp.float32)
         + jnp.dot(h_ref[1], w_ref[DH // 2:, :], preferred_element_type=jnp.float32))
    o_ref[...] = y * dis_ref[...]


def _tc_mm2(h, w2, dis_col):
    return pl.pallas_call(
        _mm2_k,
        grid=(N // _RB,),
        in_specs=[pl.BlockSpec((NC, _RB, DH // 2), lambda i: (0, i, 0)),
                  pl.BlockSpec((DH, DO), lambda i: (0, 0)),
                  pl.BlockSpec((_RB, 1), lambda i: (i, 0))],
        out_specs=pl.BlockSpec((_RB, DO), lambda i: (i, 0)),
        out_shape=jax.ShapeDtypeStruct((N, DO), jnp.float32),
    )(h, w2, dis_col)


def _soft_k(acc_ref, t2_ref, dis_ref, b2_ref, o_ref):
    z = dis_ref[...] * (acc_ref[0] + acc_ref[1] + t2_ref[...]) + b2_ref[...]
    z = z - jnp.max(z, axis=1, keepdims=True)
    ez = jnp.exp(z)
    o_ref[...] = ez / jnp.sum(ez, axis=1, keepdims=True)


def _tc_soft(acc, t2, dis_col, b2):
    return pl.pallas_call(
        _soft_k,
        grid=(N // _RB,),
        in_specs=[pl.BlockSpec((NC, _RB, DO), lambda i: (0, i, 0)),
                  pl.BlockSpec((_RB, DO), lambda i: (i, 0)),
                  pl.BlockSpec((_RB, 1), lambda i: (i, 0)),
                  pl.BlockSpec((1, DO), lambda i: (0, 0))],
        out_specs=pl.BlockSpec((_RB, DO), lambda i: (i, 0)),
        out_shape=jax.ShapeDtypeStruct((N, DO), jnp.float32),
    )(acc, t2, dis_col, b2)


def kernel(x, edge_index, edge_weight, W1, b1, W2, b2, attention):
    row = edge_index[0].astype(jnp.int32).reshape(E // W, W)
    col = edge_index[1].astype(jnp.int32).reshape(E // W, W)
    ew2 = edge_weight.reshape(E // W, W)
    p0 = jax.nn.softmax(attention, axis=0)[0]
    deg = _sc_deg(col, ew2)
    table1, dis_col = _tc_mm1(x, W1 * p0, deg.T)
    dis = dis_col[:, 0]
    h = _sc_conv(table1, row, col, ew2, b1 * p0, dis,
                 DH // 2, True, 8, False)
    table2 = _tc_mm2(h, W2, dis_col)
    acc2 = _sc_acc2(table2, row, col, ew2)
    return _tc_soft(acc2, table2, dis_col, b2[None, :])


# speedup vs baseline: 1.7750x; 1.0896x over previous
import functools

import jax
import jax.numpy as jnp
from jax import lax
from jax.experimental import pallas as pl
from jax.experimental.pallas import tpu as pltpu
from jax.experimental.pallas import tpu_sc as plsc

N = 10000
E = 320000
DH = 128
DO = 32
NC = 2
NS = 16
L = 16
NPAD = 10240
RPT = NPAD // NS
W = 125
WPT = E // NC // NS // W
WCH = 80
RCH = RPT // WCH

_MESH = plsc.VectorSubcoreMesh(core_axis_name="c", subcore_axis_name="s")


def _zero_rows(ref, nrow, ncol):
    def body(i, _):
        for j in range(ncol // L):
            ref[i, pl.ds(j * L, L)] = jnp.zeros((L,), jnp.float32)
        return 0
    lax.fori_loop(0, nrow, body, 0)


def _deg_body(coli, ew, deg_out, coli_v, ew_v, zd_v, deg_sh, sem):
    c = lax.axis_index("c")
    s = lax.axis_index("s")
    def zb(i, _):
        zd_v[pl.ds(i * L, L)] = jnp.zeros((L,), jnp.float32)
        return 0
    lax.fori_loop(0, RPT // L, zb, 0)
    pltpu.sync_copy(zd_v, deg_sh.at[pl.ds(s * RPT, RPT)])
    wbase = (c * NS + s) * WPT
    pltpu.sync_copy(coli.at[pl.ds(wbase, WPT)], coli_v)
    pltpu.sync_copy(ew.at[pl.ds(wbase, WPT)], ew_v)
    plsc.subcore_barrier()
    def grp(g, _):
        ds_ = [pltpu.async_copy(ew_v.at[g * 8 + k],
                                deg_sh.at[coli_v.at[g * 8 + k]], sem,
                                add=True)
               for k in range(8)]
        for d in ds_:
            d.wait()
        return 0
    lax.fori_loop(0, WPT // 8, grp, 0)
    plsc.subcore_barrier()
    pltpu.sync_copy(deg_sh.at[pl.ds(s * RPT, RPT)],
                    deg_out.at[c, pl.ds(s * RPT, RPT)])


def _sc_deg(coli, ew):
    f = pl.kernel(
        _deg_body,
        out_type=jax.ShapeDtypeStruct((NC, NPAD), jnp.float32),
        mesh=_MESH,
        compiler_params=pltpu.CompilerParams(needs_layout_passes=False,
                                             use_tc_tiling_on_sc=False),
        scratch_types=[
            pltpu.VMEM((WPT, W), jnp.int32),
            pltpu.VMEM((WPT, W), jnp.float32),
            pltpu.VMEM((RPT,), jnp.float32),
            pltpu.VMEM_SHARED((NPAD,), jnp.float32),
            pltpu.SemaphoreType.DMA,
        ],
    )
    return f(coli, ew)


def _acc_body(dd, gdep, bw, table, rowi, coli, ew, acc_out, *refs):
    rowi_v, coli_v, ew_v = refs[0], refs[1], refs[2]
    rbufs = refs[3:3 + gdep]
    acc_sh, gsem, ssem = refs[3 + gdep:]
    c = lax.axis_index("c")
    s = lax.axis_index("s")
    rb0 = rbufs[0]

    _zero_rows(rb0, WCH, dd)
    for k in range(RCH):
        pltpu.sync_copy(rb0.at[pl.ds(0, WCH)],
                        acc_sh.at[pl.ds(s * RPT + k * WCH, WCH)])
    plsc.subcore_barrier()

    def block(blk, _):
        wbase = (c * NS + s) * WPT + blk * bw
        pltpu.sync_copy(rowi.at[pl.ds(wbase, bw)], rowi_v)
        pltpu.sync_copy(coli.at[pl.ds(wbase, bw)], coli_v)
        pltpu.sync_copy(ew.at[pl.ds(wbase, bw)], ew_v)
        def group(g, _):
            gd = [pltpu.async_copy(table.at[rowi_v.at[g * gdep + k]],
                                   rbufs[k], gsem)
                  for k in range(gdep)]
            sd = []
            for k in range(gdep):
                w = g * gdep + k
                gd[k].wait()
                @plsc.parallel_loop(0, W, unroll=5)
                def scale(e):
                    sp = plsc.load_gather(
                        ew_v, [jnp.zeros((L,), jnp.int32) + w,
                               jnp.zeros((L,), jnp.int32) + e])
                    for j in range(dd // L):
                        rbufs[k][e, pl.ds(j * L, L)] = (
                            rbufs[k][e, pl.ds(j * L, L)] * sp)
                sd.append(pltpu.async_copy(rbufs[k], acc_sh.at[coli_v.at[w]],
                                           ssem, add=True))
            for d in sd:
                d.wait()
            return 0
        lax.fori_loop(0, bw // gdep, group, 0)
        return 0
    lax.fori_loop(0, WPT // bw, block, 0)
    plsc.subcore_barrier()
    pltpu.sync_copy(acc_sh.at[pl.ds(s * RPT, RPT)],
                    acc_out.at[c, pl.ds(s * RPT, RPT)])


def _sc_acc(table, rowi, coli, ew, dd, gdep, bw):
    scratch = [
        pltpu.VMEM((bw, W), jnp.int32),
        pltpu.VMEM((bw, W), jnp.int32),
        pltpu.VMEM((bw, W), jnp.float32),
    ]
    scratch += [pltpu.VMEM((W, dd), jnp.float32) for _ in range(gdep)]
    scratch += [
        pltpu.VMEM_SHARED((NPAD, dd), jnp.float32),
        pltpu.SemaphoreType.DMA,
        pltpu.SemaphoreType.DMA,
    ]
    f = pl.kernel(
        functools.partial(_acc_body, dd, gdep, bw),
        out_type=jax.ShapeDtypeStruct((NC, NPAD, dd), jnp.float32),
        mesh=_MESH,
        compiler_params=pltpu.CompilerParams(needs_layout_passes=False,
                                             use_tc_tiling_on_sc=False),
        scratch_types=scratch,
    )
    return f(table, rowi, coli, ew)


_RB = 1000


def _mm1_k(x_ref, w_ref, deg_ref, o_ref, dis_ref):
    dv = lax.rsqrt(deg_ref[:, 0] + deg_ref[:, 1] + 1.0)[:, None]
    y = jnp.dot(x_ref[...], w_ref[...], preferred_element_type=jnp.float32)
    o_ref[...] = y * dv
    dis_ref[...] = dv


def _tc_mm1(x, w1, deg):
    return pl.pallas_call(
        _mm1_k,
        grid=(N // _RB,),
        in_specs=[pl.BlockSpec((_RB, DH), lambda i: (i, 0)),
                  pl.BlockSpec((DH, DH), lambda i: (0, 0)),
                  pl.BlockSpec((_RB, NC), lambda i: (i, 0))],
        out_specs=[pl.BlockSpec((_RB, DH), lambda i: (i, 0)),
                   pl.BlockSpec((_RB, 1), lambda i: (i, 0))],
        out_shape=[jax.ShapeDtypeStruct((N, DH), jnp.float32),
                   jax.ShapeDtypeStruct((N, 1), jnp.float32)],
    )(x, w1, deg)


def _mm2_k(acc_ref, t1_ref, dis_ref, b1_ref, w_ref, o_ref):
    h = dis_ref[...] * (acc_ref[0] + acc_ref[1] + t1_ref[...]) + b1_ref[...]
    h = jnp.maximum(h, 0.0)
    y = jnp.dot(h, w_ref[...], preferred_element_type=jnp.float32)
    o_ref[...] = y * dis_ref[...]


def _tc_mm2(acc1, t1, dis_col, b1, w2):
    return pl.pallas_call(
        _mm2_k,
        grid=(N // _RB,),
        in_specs=[pl.BlockSpec((NC, _RB, DH), lambda i: (0, i, 0)),
                  pl.BlockSpec((_RB, DH), lambda i: (i, 0)),
                  pl.BlockSpec((_RB, 1), lambda i: (i, 0)),
                  pl.BlockSpec((1, DH), lambda i: (0, 0)),
                  pl.BlockSpec((DH, DO), lambda i: (0, 0))],
        out_specs=pl.BlockSpec((_RB, DO), lambda i: (i, 0)),
        out_shape=jax.ShapeDtypeStruct((N, DO), jnp.float32),
    )(acc1, t1, dis_col, b1, w2)


def _soft_k(acc_ref, t2_ref, dis_ref, b2_ref, o_ref):
    z = dis_ref[...] * (acc_ref[0] + acc_ref[1] + t2_ref[...]) + b2_ref[...]
    z = z - jnp.max(z, axis=1, keepdims=True)
    ez = jnp.exp(z)
    o_ref[...] = ez / jnp.sum(ez, axis=1, keepdims=True)


def _tc_soft(acc, t2, dis_col, b2):
    return pl.pallas_call(
        _soft_k,
        grid=(N // _RB,),
        in_specs=[pl.BlockSpec((NC, _RB, DO), lambda i: (0, i, 0)),
                  pl.BlockSpec((_RB, DO), lambda i: (i, 0)),
                  pl.BlockSpec((_RB, 1), lambda i: (i, 0)),
                  pl.BlockSpec((1, DO), lambda i: (0, 0))],
        out_specs=pl.BlockSpec((_RB, DO), lambda i: (i, 0)),
        out_shape=jax.ShapeDtypeStruct((N, DO), jnp.float32),
    )(acc, t2, dis_col, b2)


def kernel(x, edge_index, edge_weight, W1, b1, W2, b2, attention):
    row = edge_index[0].astype(jnp.int32).reshape(E // W, W)
    col = edge_index[1].astype(jnp.int32).reshape(E // W, W)
    ew2 = edge_weight.reshape(E // W, W)
    p0 = jax.nn.softmax(attention, axis=0)[0]
    deg = _sc_deg(col, ew2)
    table1, dis_col = _tc_mm1(x, W1 * p0, deg.T)
    acc1 = _sc_acc(table1, row, col, ew2, DH, 2, 20)
    table2 = _tc_mm2(acc1, table1, dis_col, (b1 * p0)[None, :], W2)
    acc2 = _sc_acc(table2, row, col, ew2, DO, 4, 40)
    return _tc_soft(acc2, table2, dis_col, b2[None, :])


# speedup vs baseline: 1.8323x; 1.0323x over previous
import functools

import jax
import jax.numpy as jnp
from jax import lax
from jax.experimental import pallas as pl
from jax.experimental.pallas import tpu as pltpu
from jax.experimental.pallas import tpu_sc as plsc

N = 10000
E = 320000
DH = 128
DO = 32
NC = 2
NS = 16
L = 16
NPAD = 10240
RPT = NPAD // NS
W = 125
WPT = E // NC // NS // W
WCH = 80
RCH = RPT // WCH

_MESH = plsc.VectorSubcoreMesh(core_axis_name="c", subcore_axis_name="s")


def _zero_rows(ref, nrow, ncol):
    def body(i, _):
        for j in range(ncol // L):
            ref[i, pl.ds(j * L, L)] = jnp.zeros((L,), jnp.float32)
        return 0
    lax.fori_loop(0, nrow, body, 0)


def _deg_body(coli, ew, deg_out, coli_v, ew_v, zd_v, deg_sh, sem):
    c = lax.axis_index("c")
    s = lax.axis_index("s")
    def zb(i, _):
        zd_v[pl.ds(i * L, L)] = jnp.zeros((L,), jnp.float32)
        return 0
    lax.fori_loop(0, RPT // L, zb, 0)
    pltpu.sync_copy(zd_v, deg_sh.at[pl.ds(s * RPT, RPT)])
    wbase = (c * NS + s) * WPT
    pltpu.sync_copy(coli.at[pl.ds(wbase, WPT)], coli_v)
    pltpu.sync_copy(ew.at[pl.ds(wbase, WPT)], ew_v)
    plsc.subcore_barrier()
    def grp(g, _):
        ds_ = [pltpu.async_copy(ew_v.at[g * 8 + k],
                                deg_sh.at[coli_v.at[g * 8 + k]], sem,
                                add=True)
               for k in range(8)]
        for d in ds_:
            d.wait()
        return 0
    lax.fori_loop(0, WPT // 8, grp, 0)
    plsc.subcore_barrier()
    pltpu.sync_copy(deg_sh.at[pl.ds(s * RPT, RPT)],
                    deg_out.at[c, pl.ds(s * RPT, RPT)])


def _sc_deg(coli, ew):
    f = pl.kernel(
        _deg_body,
        out_type=jax.ShapeDtypeStruct((NC, NPAD), jnp.float32),
        mesh=_MESH,
        compiler_params=pltpu.CompilerParams(needs_layout_passes=False,
                                             use_tc_tiling_on_sc=False),
        scratch_types=[
            pltpu.VMEM((WPT, W), jnp.int32),
            pltpu.VMEM((WPT, W), jnp.float32),
            pltpu.VMEM((RPT,), jnp.float32),
            pltpu.VMEM_SHARED((NPAD,), jnp.float32),
            pltpu.SemaphoreType.DMA,
        ],
    )
    return f(coli, ew)


def _acc_body(dd, gdep, bw, table, rowi, coli, ew, acc_out, *refs):
    rowi_v, coli_v, ew_v = refs[0], refs[1], refs[2]
    rbufs = refs[3:3 + gdep]
    acc_sh, gsem, ssem = refs[3 + gdep:]
    c = lax.axis_index("c")
    s = lax.axis_index("s")
    rb0 = rbufs[0]

    _zero_rows(rb0, WCH, dd)
    for k in range(RCH):
        pltpu.sync_copy(rb0.at[pl.ds(0, WCH)],
                        acc_sh.at[pl.ds(s * RPT + k * WCH, WCH)])
    plsc.subcore_barrier()

    def block(blk, _):
        wbase = (c * NS + s) * WPT + blk * bw
        pltpu.sync_copy(rowi.at[pl.ds(wbase, bw)], rowi_v)
        pltpu.sync_copy(coli.at[pl.ds(wbase, bw)], coli_v)
        pltpu.sync_copy(ew.at[pl.ds(wbase, bw)], ew_v)
        def group(g, _):
            gd = [pltpu.async_copy(table.at[rowi_v.at[g * gdep + k]],
                                   rbufs[k], gsem)
                  for k in range(gdep)]
            sd = []
            for k in range(gdep):
                w = g * gdep + k
                gd[k].wait()
                @plsc.parallel_loop(0, W, unroll=5)
                def scale(e):
                    sp = plsc.load_gather(
                        ew_v, [jnp.zeros((L,), jnp.int32) + w,
                               jnp.zeros((L,), jnp.int32) + e])
                    for j in range(dd // L):
                        rbufs[k][e, pl.ds(j * L, L)] = (
                            rbufs[k][e, pl.ds(j * L, L)] * sp)
                sd.append(pltpu.async_copy(rbufs[k], acc_sh.at[coli_v.at[w]],
                                           ssem, add=True))
            for d in sd:
                d.wait()
            return 0
        lax.fori_loop(0, bw // gdep, group, 0)
        return 0
    lax.fori_loop(0, WPT // bw, block, 0)
    plsc.subcore_barrier()
    pltpu.sync_copy(acc_sh.at[pl.ds(s * RPT, RPT)],
                    acc_out.at[c, pl.ds(s * RPT, RPT)])


def _sc_acc(table, rowi, coli, ew, dd, gdep, bw):
    scratch = [
        pltpu.VMEM((bw, W), jnp.int32),
        pltpu.VMEM((bw, W), jnp.int32),
        pltpu.VMEM((bw, W), jnp.float32),
    ]
    scratch += [pltpu.VMEM((W, dd), jnp.float32) for _ in range(gdep)]
    scratch += [
        pltpu.VMEM_SHARED((NPAD, dd), jnp.float32),
        pltpu.SemaphoreType.DMA,
        pltpu.SemaphoreType.DMA,
    ]
    f = pl.kernel(
        functools.partial(_acc_body, dd, gdep, bw),
        out_type=jax.ShapeDtypeStruct((NC, NPAD, dd), jnp.float32),
        mesh=_MESH,
        compiler_params=pltpu.CompilerParams(needs_layout_passes=False,
                                             use_tc_tiling_on_sc=False),
        scratch_types=scratch,
    )
    return f(table, rowi, coli, ew)


_RB = 1000


def _mm1_k(x_ref, w_ref, deg_ref, o_ref, dis_ref):
    dv = lax.rsqrt(deg_ref[:, 0] + deg_ref[:, 1] + 1.0)[:, None]
    y = jnp.dot(x_ref[...], w_ref[...], preferred_element_type=jnp.float32)
    o_ref[...] = y * dv
    dis_ref[...] = dv


def _tc_mm1(x, w1, deg):
    return pl.pallas_call(
        _mm1_k,
        grid=(N // _RB,),
        in_specs=[pl.BlockSpec((_RB, DH), lambda i: (i, 0)),
                  pl.BlockSpec((DH, DH), lambda i: (0, 0)),
                  pl.BlockSpec((_RB, NC), lambda i: (i, 0))],
        out_specs=[pl.BlockSpec((_RB, DH), lambda i: (i, 0)),
                   pl.BlockSpec((_RB, 1), lambda i: (i, 0))],
        out_shape=[jax.ShapeDtypeStruct((N, DH), jnp.float32),
                   jax.ShapeDtypeStruct((N, 1), jnp.float32)],
    )(x, w1, deg)


def _mm2_k(acc_ref, t1_ref, dis_ref, b1_ref, w_ref, o_ref):
    h = dis_ref[...] * (acc_ref[0] + acc_ref[1] + t1_ref[...]) + b1_ref[...]
    h = jnp.maximum(h, 0.0)
    y = jnp.dot(h, w_ref[...], preferred_element_type=jnp.float32)
    o_ref[...] = y * dis_ref[...]


def _tc_mm2(acc1, t1, dis_col, b1, w2):
    return pl.pallas_call(
        _mm2_k,
        grid=(N // _RB,),
        in_specs=[pl.BlockSpec((NC, _RB, DH), lambda i: (0, i, 0)),
                  pl.BlockSpec((_RB, DH), lambda i: (i, 0)),
                  pl.BlockSpec((_RB, 1), lambda i: (i, 0)),
                  pl.BlockSpec((1, DH), lambda i: (0, 0)),
                  pl.BlockSpec((DH, DO), lambda i: (0, 0))],
        out_specs=pl.BlockSpec((_RB, DO), lambda i: (i, 0)),
        out_shape=jax.ShapeDtypeStruct((N, DO), jnp.float32),
    )(acc1, t1, dis_col, b1, w2)


def _soft_k(acc_ref, t2_ref, dis_ref, b2_ref, o_ref):
    z = dis_ref[...] * (acc_ref[0] + acc_ref[1] + t2_ref[...]) + b2_ref[...]
    z = z - jnp.max(z, axis=1, keepdims=True)
    ez = jnp.exp(z)
    o_ref[...] = ez / jnp.sum(ez, axis=1, keepdims=True)


def _tc_soft(acc, t2, dis_col, b2):
    return pl.pallas_call(
        _soft_k,
        grid=(N // _RB,),
        in_specs=[pl.BlockSpec((NC, _RB, DO), lambda i: (0, i, 0)),
                  pl.BlockSpec((_RB, DO), lambda i: (i, 0)),
                  pl.BlockSpec((_RB, 1), lambda i: (i, 0)),
                  pl.BlockSpec((1, DO), lambda i: (0, 0))],
        out_specs=pl.BlockSpec((_RB, DO), lambda i: (i, 0)),
        out_shape=jax.ShapeDtypeStruct((N, DO), jnp.float32),
    )(acc, t2, dis_col, b2)


def kernel(x, edge_index, edge_weight, W1, b1, W2, b2, attention):
    row = edge_index[0].astype(jnp.int32).reshape(E // W, W)
    col = edge_index[1].astype(jnp.int32).reshape(E // W, W)
    ew2 = edge_weight.reshape(E // W, W)
    p0 = jax.nn.softmax(attention, axis=0)[0]
    deg = _sc_deg(col, ew2)
    table1, dis_col = _tc_mm1(x, W1 * p0, deg.T)
    acc1 = _sc_acc(table1, row, col, ew2, DH, 2, 40)
    table2 = _tc_mm2(acc1, table1, dis_col, (b1 * p0)[None, :], W2)
    acc2 = _sc_acc(table2, row, col, ew2, DO, 8, 40)
    return _tc_soft(acc2, table2, dis_col, b2[None, :])


# speedup vs baseline: 1.8421x; 1.0054x over previous
import functools

import jax
import jax.numpy as jnp
from jax import lax
from jax.experimental import pallas as pl
from jax.experimental.pallas import tpu as pltpu
from jax.experimental.pallas import tpu_sc as plsc

N = 10000
E = 320000
DH = 128
DO = 32
NC = 2
NS = 16
L = 16
NPAD = 10240
RPT = NPAD // NS
W = 125
WPT = E // NC // NS // W
WCH = 80
RCH = RPT // WCH

_MESH = plsc.VectorSubcoreMesh(core_axis_name="c", subcore_axis_name="s")


def _zero_rows(ref, nrow, ncol):
    def body(i, _):
        for j in range(ncol // L):
            ref[i, pl.ds(j * L, L)] = jnp.zeros((L,), jnp.float32)
        return 0
    lax.fori_loop(0, nrow, body, 0)


def _deg_body(coli, ew, deg_out, coli_v, ew_v, zd_v, deg_sh, sem):
    c = lax.axis_index("c")
    s = lax.axis_index("s")
    def zb(i, _):
        zd_v[pl.ds(i * L, L)] = jnp.zeros((L,), jnp.float32)
        return 0
    lax.fori_loop(0, RPT // L, zb, 0)
    pltpu.sync_copy(zd_v, deg_sh.at[pl.ds(s * RPT, RPT)])
    wbase = (c * NS + s) * WPT
    pltpu.sync_copy(coli.at[pl.ds(wbase, WPT)], coli_v)
    pltpu.sync_copy(ew.at[pl.ds(wbase, WPT)], ew_v)
    plsc.subcore_barrier()
    def grp(g, _):
        ds_ = [pltpu.async_copy(ew_v.at[g * 8 + k],
                                deg_sh.at[coli_v.at[g * 8 + k]], sem,
                                add=True)
               for k in range(8)]
        for d in ds_:
            d.wait()
        return 0
    lax.fori_loop(0, WPT // 8, grp, 0)
    plsc.subcore_barrier()
    pltpu.sync_copy(deg_sh.at[pl.ds(s * RPT, RPT)],
                    deg_out.at[c, pl.ds(s * RPT, RPT)])


def _sc_deg(coli, ew):
    f = pl.kernel(
        _deg_body,
        out_type=jax.ShapeDtypeStruct((NC, NPAD), jnp.float32),
        mesh=_MESH,
        compiler_params=pltpu.CompilerParams(needs_layout_passes=False,
                                             use_tc_tiling_on_sc=False),
        scratch_types=[
            pltpu.VMEM((WPT, W), jnp.int32),
            pltpu.VMEM((WPT, W), jnp.float32),
            pltpu.VMEM((RPT,), jnp.float32),
            pltpu.VMEM_SHARED((NPAD,), jnp.float32),
            pltpu.SemaphoreType.DMA,
        ],
    )
    return f(coli, ew)


def _acc_body(dd, gdep, bw, table, rowi, coli, ew, acc_out, *refs):
    rowi_v, coli_v, ew_v = refs[0], refs[1], refs[2]
    rbufs = refs[3:3 + gdep]
    acc_sh, gsem, ssem = refs[3 + gdep:]
    c = lax.axis_index("c")
    s = lax.axis_index("s")
    rb0 = rbufs[0]

    _zero_rows(rb0, WCH, dd)
    for k in range(RCH):
        pltpu.sync_copy(rb0.at[pl.ds(0, WCH)],
                        acc_sh.at[pl.ds(s * RPT + k * WCH, WCH)])
    plsc.subcore_barrier()

    def block(blk, _):
        wbase = (c * NS + s) * WPT + blk * bw
        pltpu.sync_copy(rowi.at[pl.ds(wbase, bw)], rowi_v)
        pltpu.sync_copy(coli.at[pl.ds(wbase, bw)], coli_v)
        pltpu.sync_copy(ew.at[pl.ds(wbase, bw)], ew_v)
        def group(g, _):
            gd = [pltpu.async_copy(table.at[rowi_v.at[g * gdep + k]],
                                   rbufs[k], gsem)
                  for k in range(gdep)]
            sd = []
            for k in range(gdep):
                w = g * gdep + k
                gd[k].wait()
                @plsc.parallel_loop(0, W, unroll=5)
                def scale(e):
                    sp = plsc.load_gather(
                        ew_v, [jnp.zeros((L,), jnp.int32) + w,
                               jnp.zeros((L,), jnp.int32) + e])
                    for j in range(dd // L):
                        rbufs[k][e, pl.ds(j * L, L)] = (
                            rbufs[k][e, pl.ds(j * L, L)] * sp)
                sd.append(pltpu.async_copy(rbufs[k], acc_sh.at[coli_v.at[w]],
                                           ssem, add=True))
            for d in sd:
                d.wait()
            return 0
        lax.fori_loop(0, bw // gdep, group, 0)
        return 0
    lax.fori_loop(0, WPT // bw, block, 0)
    plsc.subcore_barrier()
    pltpu.sync_copy(acc_sh.at[pl.ds(s * RPT, RPT)],
                    acc_out.at[c, pl.ds(s * RPT, RPT)])


def _sc_acc(table, rowi, coli, ew, dd, gdep, bw):
    scratch = [
        pltpu.VMEM((bw, W), jnp.int32),
        pltpu.VMEM((bw, W), jnp.int32),
        pltpu.VMEM((bw, W), jnp.float32),
    ]
    scratch += [pltpu.VMEM((W, dd), jnp.float32) for _ in range(gdep)]
    scratch += [
        pltpu.VMEM_SHARED((NPAD, dd), jnp.float32),
        pltpu.SemaphoreType.DMA,
        pltpu.SemaphoreType.DMA,
    ]
    f = pl.kernel(
        functools.partial(_acc_body, dd, gdep, bw),
        out_type=jax.ShapeDtypeStruct((NC, NPAD, dd), jnp.float32),
        mesh=_MESH,
        compiler_params=pltpu.CompilerParams(needs_layout_passes=False,
                                             use_tc_tiling_on_sc=False),
        scratch_types=scratch,
    )
    return f(table, rowi, coli, ew)


_RB = 1000


def _mm1_k(x_ref, w_ref, deg_ref, o_ref, dis_ref):
    dv = lax.rsqrt(deg_ref[:, 0] + deg_ref[:, 1] + 1.0)[:, None]
    y = jnp.dot(x_ref[...], w_ref[...], preferred_element_type=jnp.float32)
    o_ref[...] = y * dv
    dis_ref[...] = dv


def _tc_mm1(x, w1, deg):
    return pl.pallas_call(
        _mm1_k,
        grid=(N // _RB,),
        in_specs=[pl.BlockSpec((_RB, DH), lambda i: (i, 0)),
                  pl.BlockSpec((DH, DH), lambda i: (0, 0)),
                  pl.BlockSpec((_RB, NC), lambda i: (i, 0))],
        out_specs=[pl.BlockSpec((_RB, DH), lambda i: (i, 0)),
                   pl.BlockSpec((_RB, 1), lambda i: (i, 0))],
        out_shape=[jax.ShapeDtypeStruct((N, DH), jnp.float32),
                   jax.ShapeDtypeStruct((N, 1), jnp.float32)],
    )(x, w1, deg)


def _mm2_k(acc_ref, t1_ref, dis_ref, b1_ref, w_ref, o_ref):
    h = dis_ref[...] * (acc_ref[0] + acc_ref[1] + t1_ref[...]) + b1_ref[...]
    h = jnp.maximum(h, 0.0)
    y = jnp.dot(h, w_ref[...], preferred_element_type=jnp.float32)
    o_ref[...] = y * dis_ref[...]


def _tc_mm2(acc1, t1, dis_col, b1, w2):
    return pl.pallas_call(
        _mm2_k,
        grid=(N // _RB,),
        in_specs=[pl.BlockSpec((NC, _RB, DH), lambda i: (0, i, 0)),
                  pl.BlockSpec((_RB, DH), lambda i: (i, 0)),
                  pl.BlockSpec((_RB, 1), lambda i: (i, 0)),
                  pl.BlockSpec((1, DH), lambda i: (0, 0)),
                  pl.BlockSpec((DH, DO), lambda i: (0, 0))],
        out_specs=pl.BlockSpec((_RB, DO), lambda i: (i, 0)),
        out_shape=jax.ShapeDtypeStruct((N, DO), jnp.float32),
    )(acc1, t1, dis_col, b1, w2)


def _soft_k(acc_ref, t2_ref, dis_ref, b2_ref, o_ref):
    z = dis_ref[...] * (acc_ref[0] + acc_ref[1] + t2_ref[...]) + b2_ref[...]
    z = z - jnp.max(z, axis=1, keepdims=True)
    ez = jnp.exp(z)
    o_ref[...] = ez / jnp.sum(ez, axis=1, keepdims=True)


def _tc_soft(acc, t2, dis_col, b2):
    return pl.pallas_call(
        _soft_k,
        grid=(N // _RB,),
        in_specs=[pl.BlockSpec((NC, _RB, DO), lambda i: (0, i, 0)),
                  pl.BlockSpec((_RB, DO), lambda i: (i, 0)),
                  pl.BlockSpec((_RB, 1), lambda i: (i, 0)),
                  pl.BlockSpec((1, DO), lambda i: (0, 0))],
        out_specs=pl.BlockSpec((_RB, DO), lambda i: (i, 0)),
        out_shape=jax.ShapeDtypeStruct((N, DO), jnp.float32),
    )(acc, t2, dis_col, b2)


def kernel(x, edge_index, edge_weight, W1, b1, W2, b2, attention):
    row = edge_index[0].astype(jnp.int32).reshape(E // W, W)
    col = edge_index[1].astype(jnp.int32).reshape(E // W, W)
    ew2 = edge_weight.reshape(E // W, W)
    p0 = jax.nn.softmax(attention, axis=0)[0]
    deg = _sc_deg(col, ew2)
    table1, dis_col = _tc_mm1(x, W1 * p0, deg.T)
    acc1 = _sc_acc(table1, row, col, ew2, DH, 2, 40)
    table2 = _tc_mm2(acc1, table1, dis_col, (b1 * p0)[None, :], W2)
    acc2 = _sc_acc(table2, row, col, ew2, DO, 8, 80)
    return _tc_soft(acc2, table2, dis_col, b2[None, :])
